# baseline (speedup 1.0000x reference)
import jax
import jax.numpy as jnp
from jax.experimental import pallas as pl
from jax.experimental.pallas import tpu as pltpu

N_TOKENS = 16384
IN_DIM = 1024
HIDDEN = 256
NUM_EXPERTS = 16

BM = 2048
NSTREAM = 2


def _dot(a, b):
    return jax.lax.dot_general(
        a, b, (((1,), (0,)), ((), ())), preferred_element_type=jnp.float32
    )


def _route_tile(x, w1, b1, w2, b2, lt, iota_row):
    h = _dot(x.astype(jnp.bfloat16), w1)
    h = jnp.maximum(h + b1, 0.0)
    logits = _dot(h.astype(jnp.bfloat16), w2)
    logits = logits + b2
    m = jnp.max(logits, axis=-1, keepdims=True)
    e = jnp.exp(logits - m)
    probs = e / jnp.sum(e, axis=-1, keepdims=True)
    probs_t = jax.lax.transpose(probs, (1, 0))

    mask = (logits == m).astype(jnp.bfloat16)
    cnt = _dot(mask, lt)
    first = jnp.where(cnt == 1.0, mask.astype(jnp.float32), 0.0).astype(jnp.bfloat16)
    first_t = jax.lax.transpose(first, (1, 0))
    eid_lane = _dot(iota_row, first_t)
    return probs_t, eid_lane.astype(jnp.int32).reshape(BM)


def _router_body(x0_ref, x1_ref, w1_ref, b1_ref, w2_ref, b2_ref, p_ref, e_ref):
    w1 = w1_ref[...].astype(jnp.bfloat16)
    b1 = b1_ref[...].reshape(1, HIDDEN)
    w2 = w2_ref[...].astype(jnp.bfloat16)
    b2 = b2_ref[...].reshape(1, NUM_EXPERTS)
    lt = (
        jax.lax.broadcasted_iota(jnp.int32, (NUM_EXPERTS, NUM_EXPERTS), 0)
        <= jax.lax.broadcasted_iota(jnp.int32, (NUM_EXPERTS, NUM_EXPERTS), 1)
    ).astype(jnp.bfloat16)
    iota_row = jax.lax.broadcasted_iota(
        jnp.int32, (1, NUM_EXPERTS), 1
    ).astype(jnp.bfloat16)
    p0, e0 = _route_tile(x0_ref[...], w1, b1, w2, b2, lt, iota_row)
    p1, e1 = _route_tile(x1_ref[...], w1, b1, w2, b2, lt, iota_row)
    p_ref[:, :BM] = p0
    p_ref[:, BM:] = p1
    e_ref[:BM] = e0
    e_ref[BM:] = e1


def kernel(patch_feat, W1, b1, W2, b2):
    grid = (N_TOKENS // (BM * NSTREAM),)
    probs_t, eid = pl.pallas_call(
        _router_body,
        grid=grid,
        in_specs=[
            pl.BlockSpec((BM, IN_DIM), lambda i: (2 * i, 0)),
            pl.BlockSpec((BM, IN_DIM), lambda i: (2 * i + 1, 0)),
            pl.BlockSpec((IN_DIM, HIDDEN), lambda i: (0, 0)),
            pl.BlockSpec((HIDDEN,), lambda i: (0,)),
            pl.BlockSpec((HIDDEN, NUM_EXPERTS), lambda i: (0, 0)),
            pl.BlockSpec((NUM_EXPERTS,), lambda i: (0,)),
        ],
        out_specs=[
            pl.BlockSpec((NUM_EXPERTS, BM * NSTREAM), lambda i: (0, i)),
            pl.BlockSpec((BM * NSTREAM,), lambda i: (i,)),
        ],
        out_shape=[
            jax.ShapeDtypeStruct((NUM_EXPERTS, N_TOKENS), jnp.float32),
            jax.ShapeDtypeStruct((N_TOKENS,), jnp.int32),
        ],
    )(patch_feat, patch_feat, W1, b1, W2, b2)
    return probs_t.T, eid


# speedup vs baseline: 1.1192x; 1.1192x over previous
import jax
import jax.numpy as jnp
from jax.experimental import pallas as pl
from jax.experimental.pallas import tpu as pltpu

N_TOKENS = 16384
IN_DIM = 1024
HIDDEN = 256
NUM_EXPERTS = 16

BM = 4096


def _dot(a, b):
    return jax.lax.dot_general(
        a, b, (((1,), (0,)), ((), ())), preferred_element_type=jnp.float32
    )


def _route_tile(x, w1, b1, w2, b2, lt, iota_row):
    h = _dot(x.astype(jnp.bfloat16), w1)
    h = jnp.maximum(h + b1, 0.0)
    logits = _dot(h.astype(jnp.bfloat16), w2)
    logits = logits + b2
    m = jnp.max(logits, axis=-1, keepdims=True)
    e = jnp.exp(logits - m)
    probs = e / jnp.sum(e, axis=-1, keepdims=True)
    probs_t = jax.lax.transpose(probs, (1, 0))

    mask = (logits == m).astype(jnp.bfloat16)
    cnt = _dot(mask, lt)
    first = jnp.where(cnt == 1.0, mask.astype(jnp.float32), 0.0).astype(jnp.bfloat16)
    first_t = jax.lax.transpose(first, (1, 0))
    eid_lane = _dot(iota_row, first_t)
    return probs_t, eid_lane.astype(jnp.int32).reshape(BM)


def _router_body(x_ref, w1_ref, b1_ref, w2_ref, b2_ref, p_ref, e_ref):
    w1 = w1_ref[...].astype(jnp.bfloat16)
    b1 = b1_ref[...].reshape(1, HIDDEN)
    w2 = w2_ref[...].astype(jnp.bfloat16)
    b2 = b2_ref[...].reshape(1, NUM_EXPERTS)
    lt = (
        jax.lax.broadcasted_iota(jnp.int32, (NUM_EXPERTS, NUM_EXPERTS), 0)
        <= jax.lax.broadcasted_iota(jnp.int32, (NUM_EXPERTS, NUM_EXPERTS), 1)
    ).astype(jnp.bfloat16)
    iota_row = jax.lax.broadcasted_iota(
        jnp.int32, (1, NUM_EXPERTS), 1
    ).astype(jnp.bfloat16)
    p_ref[...], e_ref[...] = _route_tile(
        x_ref[...], w1, b1, w2, b2, lt, iota_row)


def kernel(patch_feat, W1, b1, W2, b2):
    grid = (N_TOKENS // BM,)
    probs_t, eid = pl.pallas_call(
        _router_body,
        grid=grid,
        in_specs=[
            pl.BlockSpec((BM, IN_DIM), lambda i: (i, 0)),
            pl.BlockSpec((IN_DIM, HIDDEN), lambda i: (0, 0)),
            pl.BlockSpec((HIDDEN,), lambda i: (0,)),
            pl.BlockSpec((HIDDEN, NUM_EXPERTS), lambda i: (0, 0)),
            pl.BlockSpec((NUM_EXPERTS,), lambda i: (0,)),
        ],
        out_specs=[
            pl.BlockSpec((NUM_EXPERTS, BM), lambda i: (0, i)),
            pl.BlockSpec((BM,), lambda i: (i,)),
        ],
        out_shape=[
            jax.ShapeDtypeStruct((NUM_EXPERTS, N_TOKENS), jnp.float32),
            jax.ShapeDtypeStruct((N_TOKENS,), jnp.int32),
        ],
    )(patch_feat, W1, b1, W2, b2)
    return probs_t.T, eid
